# baseline (speedup 1.0000x reference)
import jax
import jax.numpy as jnp
from jax import lax
from jax.experimental import pallas as pl
from jax.experimental.pallas import tpu as pltpu
from jax.experimental.pallas import tpu_sc as plsc

VOCAB = 100000
D = 128
B = 16384
NW = 32
ROWS_PER_W = B // NW
CHUNK = 128
NCHUNK = ROWS_PER_W // CHUNK
NSEG = D // 16


def _lane_sum(v):
    lanes = lax.iota(jnp.int32, 16)
    dnums = lax.GatherDimensionNumbers(
        offset_dims=(), collapsed_slice_dims=(0,), start_index_map=(0,))
    for sh in (8, 4, 2, 1):
        perm = lax.gather(v, (lanes ^ sh)[:, None], dnums, (1,),
                          mode=lax.GatherScatterMode.PROMISE_IN_BOUNDS)
        v = v + perm
    return v


def _rsqrt_vec(s):
    i = lax.bitcast_convert_type(s, jnp.int32)
    i = jnp.int32(0x5F3759DF) - (i >> 1)
    y = lax.bitcast_convert_type(i, jnp.float32)
    half = s * 0.5
    for _ in range(3):
        y = y * (1.5 - half * y * y)
    return y


def _sc_body(rep_hbm, idx_hbm, table_hbm, part_hbm, outr_hbm, outc_hbm,
             idx_v, rep_v, comp_v, acc_v, sem):
    wid = lax.axis_index("s") * 2 + lax.axis_index("c")
    base = wid * ROWS_PER_W

    pltpu.sync_copy(idx_hbm.at[wid], idx_v)

    err_acc = jnp.zeros((16,), jnp.float32)
    for j in range(NCHUNK):
        off = base + j * CHUNK
        pltpu.async_copy(table_hbm.at[idx_v.at[j]], comp_v, sem).wait()
        pltpu.sync_copy(rep_hbm.at[pl.ds(off, CHUNK)], rep_v)

        def row_body(r, acc):
            vr = [rep_v[r, pl.ds(16 * k, 16)] for k in range(NSEG)]
            vc = [comp_v[r, pl.ds(16 * k, 16)] for k in range(NSEG)]
            sr_v = vr[0] * vr[0]
            sc_v = vc[0] * vc[0]
            for k in range(1, NSEG):
                sr_v = sr_v + vr[k] * vr[k]
                sc_v = sc_v + vc[k] * vc[k]
            for k in range(NSEG):
                d = vc[k] - vr[k]
                acc = acc + d * d
            rr = _rsqrt_vec(jnp.maximum(_lane_sum(sr_v), 1e-24))
            rc = _rsqrt_vec(jnp.maximum(_lane_sum(sc_v), 1e-24))
            for k in range(NSEG):
                rep_v[r, pl.ds(16 * k, 16)] = vr[k] * rr
                comp_v[r, pl.ds(16 * k, 16)] = vc[k] * rc
            return acc

        err_acc = lax.fori_loop(0, CHUNK, row_body, err_acc)
        pltpu.sync_copy(rep_v, outr_hbm.at[pl.ds(off, CHUNK)])
        pltpu.sync_copy(comp_v, outc_hbm.at[pl.ds(off, CHUNK)])

    acc_v[...] = err_acc
    pltpu.sync_copy(acc_v, part_hbm.at[wid])


@jax.jit
def kernel(rep, expr, emb_weight):
    idx = expr.astype(jnp.int32).reshape(NW, NCHUNK, CHUNK)
    mesh = plsc.VectorSubcoreMesh(core_axis_name="c", subcore_axis_name="s")
    part, out_rep, out_comp = pl.kernel(
        _sc_body,
        out_type=(
            jax.ShapeDtypeStruct((NW, 16), jnp.float32),
            jax.ShapeDtypeStruct((B, D), jnp.float32),
            jax.ShapeDtypeStruct((B, D), jnp.float32),
        ),
        mesh=mesh,
        scratch_types=[
            pltpu.VMEM((NCHUNK, CHUNK), jnp.int32),
            pltpu.VMEM((CHUNK, D), jnp.float32),
            pltpu.VMEM((CHUNK, D), jnp.float32),
            pltpu.VMEM((16,), jnp.float32),
            pltpu.SemaphoreType.DMA,
        ],
    )(rep, idx, emb_weight)
    err = jnp.sum(part) * (1.0 / (B * D))
    return (err, out_rep, out_comp)
